# Initial kernel scaffold; baseline (speedup 1.0000x reference)
import functools

import jax
import jax.numpy as jnp
from jax import lax
from jax.experimental import pallas as pl
from jax.experimental.pallas import tpu as pltpu
from jax.experimental.pallas import tpu_sc as plsc

_B, _F, _V, _D = 4096, 26, 100000, 32
_DENSE_IN = 13
_FP = 32
_VBLK = 5000
_BBLK = 512
_NTILES = 32
_BCHUNK = _B // _NTILES


def _project_body(wft_ref, emb_ref, out_ref):
    out_ref[...] = jnp.dot(emb_ref[...], wft_ref[...],
                           preferred_element_type=jnp.float32)


def _project_table(emb_table, wft):
    return pl.pallas_call(
        _project_body,
        grid=(_V // _VBLK,),
        in_specs=[
            pl.BlockSpec((_D, _FP), lambda i: (0, 0)),
            pl.BlockSpec((_VBLK, _D), lambda i: (i, 0)),
        ],
        out_specs=pl.BlockSpec((_VBLK, _FP), lambda i: (i, 0)),
        out_shape=jax.ShapeDtypeStruct((_V, _FP), jnp.float32),
    )(wft, emb_table)


def _mlp_body(x_ref, w1_ref, b1_ref, w2_ref, b2_ref, w3_ref, b3_ref,
              woh_ref, bo_ref, out_ref):
    h = jnp.maximum(jnp.dot(x_ref[...], w1_ref[...],
                            preferred_element_type=jnp.float32) + b1_ref[...], 0.0)
    h = jnp.maximum(jnp.dot(h, w2_ref[...],
                            preferred_element_type=jnp.float32) + b2_ref[...], 0.0)
    h = jnp.maximum(jnp.dot(h, w3_ref[...],
                            preferred_element_type=jnp.float32) + b3_ref[...], 0.0)
    out_ref[...] = jnp.sum(h * woh_ref[...], axis=1) + bo_ref[0, 0]


def _dense_tower(x, W1, b1, W2, b2, W3, b3, woh, bo):
    full = lambda shape: pl.BlockSpec(shape, lambda i: (0,) * len(shape))
    return pl.pallas_call(
        _mlp_body,
        grid=(_B // _BBLK,),
        in_specs=[
            pl.BlockSpec((_BBLK, _DENSE_IN), lambda i: (i, 0)),
            full((_DENSE_IN, 512)), full((1, 512)),
            full((512, 256)), full((1, 256)),
            full((256, _D)), full((1, _D)),
            full((1, _D)), full((1, 1)),
        ],
        out_specs=pl.BlockSpec((_BBLK,), lambda i: (i,)),
        out_shape=jax.ShapeDtypeStruct((_B,), jnp.float32),
    )(x, W1, b1, W2, b2, W3, b3, woh, bo)


def _sc_gather_body(adj_hbm, ts_hbm, hsum_hbm, out_hbm, idx_v, g_v, h_v, o_v, sem):
    w = lax.axis_index("s") * 2 + lax.axis_index("c")
    b0 = w * _BCHUNK
    pltpu.sync_copy(adj_hbm.at[w], idx_v)
    pltpu.sync_copy(hsum_hbm.at[pl.ds(b0, _BCHUNK)], h_v)
    for half in range(2):
        cps = [
            pltpu.async_copy(ts_hbm.at[idx_v.at[half * 13 + f]],
                             g_v.at[half * 13 + f], sem)
            for f in range(13)
        ]
        for cp in cps:
            cp.wait()
    for i in range(_BCHUNK // 16):
        sl = pl.ds(i * 16, 16)
        acc = h_v[sl]
        for f in range(_F):
            acc = acc + g_v[f, sl]
        o_v[sl] = acc
    pltpu.sync_copy(o_v, out_hbm.at[pl.ds(b0, _BCHUNK)])


_sc_gather = functools.partial(
    pl.kernel,
    _sc_gather_body,
    out_type=jax.ShapeDtypeStruct((_B,), jnp.float32),
    mesh=plsc.VectorSubcoreMesh(core_axis_name="c", subcore_axis_name="s"),
    scratch_types=[
        pltpu.VMEM((_F, _BCHUNK), jnp.int32),
        pltpu.VMEM((_F, _BCHUNK), jnp.float32),
        pltpu.VMEM((_BCHUNK,), jnp.float32),
        pltpu.VMEM((_BCHUNK,), jnp.float32),
        pltpu.SemaphoreType.DMA,
    ],
)


def kernel(dense_features, sparse_indices, emb_table, W1, b1, W2, b2, W3, b3,
           Wo, bo):
    wf = Wo[: _F * _D, 0].reshape(_F, _D)
    wft = jnp.zeros((_D, _FP), jnp.float32).at[:, :_F].set(wf.T)
    woh = Wo[_F * _D:, 0].reshape(1, _D)

    ts = _project_table(emb_table, wft)
    hsum = _dense_tower(dense_features, W1, b1.reshape(1, 512), W2,
                        b2.reshape(1, 256), W3, b3.reshape(1, _D), woh,
                        bo.reshape(1, 1))

    adj = sparse_indices * _FP + jnp.arange(_F, dtype=jnp.int32)[None, :]
    adj3 = adj.reshape(_NTILES, _BCHUNK, _F).transpose(0, 2, 1)

    out = _sc_gather()(adj3, ts.reshape(_V * _FP), hsum)
    return out.reshape(_B, 1)


# speedup vs baseline: 3.8559x; 3.8559x over previous
import functools

import jax
import jax.numpy as jnp
from jax import lax
from jax.experimental import pallas as pl
from jax.experimental.pallas import tpu as pltpu
from jax.experimental.pallas import tpu_sc as plsc

_B, _F, _V, _D = 4096, 26, 100000, 32
_DENSE_IN = 13
_FP = 32
_VBLK = 5000
_BBLK = 512
_NTILES = 32
_BCHUNK = _B // _NTILES


def _project_body(wft_ref, emb_ref, out_ref):
    out_ref[...] = jnp.dot(emb_ref[...], wft_ref[...],
                           preferred_element_type=jnp.float32)


def _project_table(emb_table, wft):
    return pl.pallas_call(
        _project_body,
        grid=(_V // _VBLK,),
        in_specs=[
            pl.BlockSpec((_D, _FP), lambda i: (0, 0)),
            pl.BlockSpec((_VBLK, _D), lambda i: (i, 0)),
        ],
        out_specs=pl.BlockSpec((_VBLK, _FP), lambda i: (i, 0)),
        out_shape=jax.ShapeDtypeStruct((_V, _FP), jnp.float32),
    )(wft, emb_table)


def _mlp_body(x_ref, w1_ref, b1_ref, w2_ref, b2_ref, w3_ref, b3_ref,
              woh_ref, bo_ref, out_ref):
    h = jnp.maximum(jnp.dot(x_ref[...], w1_ref[...],
                            preferred_element_type=jnp.float32) + b1_ref[...], 0.0)
    h = jnp.maximum(jnp.dot(h, w2_ref[...],
                            preferred_element_type=jnp.float32) + b2_ref[...], 0.0)
    h = jnp.maximum(jnp.dot(h, w3_ref[...],
                            preferred_element_type=jnp.float32) + b3_ref[...], 0.0)
    out_ref[...] = jnp.sum(h * woh_ref[...], axis=1) + bo_ref[0, 0]


def _dense_tower(x, W1, b1, W2, b2, W3, b3, woh, bo):
    full = lambda shape: pl.BlockSpec(shape, lambda i: (0,) * len(shape))
    return pl.pallas_call(
        _mlp_body,
        grid=(_B // _BBLK,),
        in_specs=[
            pl.BlockSpec((_BBLK, _DENSE_IN), lambda i: (i, 0)),
            full((_DENSE_IN, 512)), full((1, 512)),
            full((512, 256)), full((1, 256)),
            full((256, _D)), full((1, _D)),
            full((1, _D)), full((1, 1)),
        ],
        out_specs=pl.BlockSpec((_BBLK,), lambda i: (i,)),
        out_shape=jax.ShapeDtypeStruct((_B,), jnp.float32),
    )(x, W1, b1, W2, b2, W3, b3, woh, bo)


def _sc_gather_body(adj_hbm, ts_hbm, hsum_hbm, out_hbm, idx_v, g_v, h_v, o_v, sem):
    w = lax.axis_index("s") * 2 + lax.axis_index("c")
    b0 = w * _BCHUNK
    pltpu.sync_copy(adj_hbm.at[w], idx_v)
    pltpu.sync_copy(hsum_hbm.at[pl.ds(b0, _BCHUNK)], h_v)
    for half in range(2):
        cps = [
            pltpu.async_copy(ts_hbm.at[idx_v.at[half * 13 + f]],
                             g_v.at[half * 13 + f], sem)
            for f in range(13)
        ]
        for cp in cps:
            cp.wait()
    for i in range(_BCHUNK // 16):
        sl = pl.ds(i * 16, 16)
        acc = h_v[sl]
        for f in range(_F):
            acc = acc + g_v[f, sl]
        o_v[sl] = acc
    pltpu.sync_copy(o_v, out_hbm.at[pl.ds(b0, _BCHUNK)])


def _sc_gather():
    return pl.kernel(
        _sc_gather_body,
        out_type=jax.ShapeDtypeStruct((_B,), jnp.float32),
        mesh=plsc.VectorSubcoreMesh(core_axis_name="c", subcore_axis_name="s",
                                    num_cores=2, num_subcores=16),
        scratch_types=[
            pltpu.VMEM((_F, _BCHUNK), jnp.int32),
            pltpu.VMEM((_F, _BCHUNK), jnp.float32),
            pltpu.VMEM((_BCHUNK,), jnp.float32),
            pltpu.VMEM((_BCHUNK,), jnp.float32),
            pltpu.SemaphoreType.DMA,
        ],
    )


def kernel(dense_features, sparse_indices, emb_table, W1, b1, W2, b2, W3, b3,
           Wo, bo):
    wf = Wo[: _F * _D, 0].reshape(_F, _D)
    wft = jnp.zeros((_D, _FP), jnp.float32).at[:, :_F].set(wf.T)
    woh = Wo[_F * _D:, 0].reshape(1, _D)

    ts = _project_table(emb_table, wft)
    hsum = _dense_tower(dense_features, W1, b1.reshape(1, 512), W2,
                        b2.reshape(1, 256), W3, b3.reshape(1, _D), woh,
                        bo.reshape(1, 1))

    adj = sparse_indices * _FP + jnp.arange(_F, dtype=jnp.int32)[None, :]
    adj3 = adj.reshape(_NTILES, _BCHUNK, _F).transpose(0, 2, 1)

    out = _sc_gather()(adj3, ts.reshape(_V * _FP), hsum)
    return out.reshape(_B, 1)


# speedup vs baseline: 6.0423x; 1.5671x over previous
import functools

import jax
import jax.numpy as jnp
from jax import lax
from jax.experimental import pallas as pl
from jax.experimental.pallas import tpu as pltpu
from jax.experimental.pallas import tpu_sc as plsc

_B, _F, _V, _D = 4096, 26, 100000, 32
_DENSE_IN = 13
_FP = 32
_VBLK = 5000
_BBLK = 2048
_NTILES = 32
_BCHUNK = _B // _NTILES


_V4 = _V // 4
_V4BLK = 5000


def _project_body(w4_ref, emb_ref, out_ref):
    out_ref[...] = jnp.dot(emb_ref[...], w4_ref[...],
                           preferred_element_type=jnp.float32)


def _project_table(emb4, w4):
    return pl.pallas_call(
        _project_body,
        grid=(_V4 // _V4BLK,),
        in_specs=[
            pl.BlockSpec((4 * _D, 4 * _FP), lambda i: (0, 0)),
            pl.BlockSpec((_V4BLK, 4 * _D), lambda i: (i, 0)),
        ],
        out_specs=pl.BlockSpec((_V4BLK, 4 * _FP), lambda i: (i, 0)),
        out_shape=jax.ShapeDtypeStruct((_V4, 4 * _FP), jnp.float32),
    )(w4, emb4)


def _mlp_body(x_ref, w1_ref, b1_ref, w2_ref, b2_ref, w3_ref, b3_ref,
              woh_ref, bo_ref, out_ref):
    h = jnp.maximum(jnp.dot(x_ref[...], w1_ref[...],
                            preferred_element_type=jnp.float32) + b1_ref[...], 0.0)
    h = jnp.maximum(jnp.dot(h, w2_ref[...],
                            preferred_element_type=jnp.float32) + b2_ref[...], 0.0)
    h = jnp.maximum(jnp.dot(h, w3_ref[...],
                            preferred_element_type=jnp.float32) + b3_ref[...], 0.0)
    out_ref[...] = jnp.sum(h * woh_ref[...], axis=1) + bo_ref[0, 0]


def _dense_tower(x, W1, b1, W2, b2, W3, b3, woh, bo):
    full = lambda shape: pl.BlockSpec(shape, lambda i: (0,) * len(shape))
    return pl.pallas_call(
        _mlp_body,
        grid=(_B // _BBLK,),
        in_specs=[
            pl.BlockSpec((_BBLK, _DENSE_IN), lambda i: (i, 0)),
            full((_DENSE_IN, 512)), full((1, 512)),
            full((512, 256)), full((1, 256)),
            full((256, _D)), full((1, _D)),
            full((1, _D)), full((1, 1)),
        ],
        out_specs=pl.BlockSpec((_BBLK,), lambda i: (i,)),
        out_shape=jax.ShapeDtypeStruct((_B,), jnp.float32),
    )(x, W1, b1, W2, b2, W3, b3, woh, bo)


def _sc_gather_body(adj_hbm, ts_hbm, hsum_hbm, out_hbm, idx_v, g_v, h_v, o_v, sem):
    w = lax.axis_index("s") * 2 + lax.axis_index("c")
    b0 = w * _BCHUNK
    pltpu.sync_copy(adj_hbm.at[w], idx_v)
    pltpu.sync_copy(hsum_hbm.at[pl.ds(b0, _BCHUNK)], h_v)
    for half in range(2):
        cps = [
            pltpu.async_copy(ts_hbm.at[idx_v.at[half * 13 + f]],
                             g_v.at[half * 13 + f], sem)
            for f in range(13)
        ]
        for cp in cps:
            cp.wait()
    for i in range(_BCHUNK // 16):
        sl = pl.ds(i * 16, 16)
        acc = h_v[sl]
        for f in range(_F):
            acc = acc + g_v[f, sl]
        o_v[sl] = acc
    pltpu.sync_copy(o_v, out_hbm.at[pl.ds(b0, _BCHUNK)])


def _sc_gather():
    return pl.kernel(
        _sc_gather_body,
        out_type=jax.ShapeDtypeStruct((_B,), jnp.float32),
        mesh=plsc.VectorSubcoreMesh(core_axis_name="c", subcore_axis_name="s",
                                    num_cores=2, num_subcores=16),
        scratch_types=[
            pltpu.VMEM((_F, _BCHUNK), jnp.int32),
            pltpu.VMEM((_F, _BCHUNK), jnp.float32),
            pltpu.VMEM((_BCHUNK,), jnp.float32),
            pltpu.VMEM((_BCHUNK,), jnp.float32),
            pltpu.SemaphoreType.DMA,
        ],
    )


def kernel(dense_features, sparse_indices, emb_table, W1, b1, W2, b2, W3, b3,
           Wo, bo):
    wf = Wo[: _F * _D, 0].reshape(_F, _D)
    wft = jnp.zeros((_D, _FP), jnp.float32).at[:, :_F].set(wf.T)
    w4 = jnp.kron(jnp.eye(4, dtype=jnp.float32), wft)
    woh = Wo[_F * _D:, 0].reshape(1, _D)

    ts = _project_table(emb_table.reshape(_V4, 4 * _D), w4)
    hsum = _dense_tower(dense_features, W1, b1.reshape(1, 512), W2,
                        b2.reshape(1, 256), W3, b3.reshape(1, _D), woh,
                        bo.reshape(1, 1))

    adj = sparse_indices * _FP + jnp.arange(_F, dtype=jnp.int32)[None, :]
    adj3 = adj.reshape(_NTILES, _BCHUNK, _F).transpose(0, 2, 1)

    out = _sc_gather()(adj3, ts.reshape(_V * _FP), hsum)
    return out.reshape(_B, 1)
